# Initial kernel scaffold; baseline (speedup 1.0000x reference)
import jax
import jax.numpy as jnp
from jax.experimental import pallas as pl


def kernel(feat, edge_index, W, b):
    raise NotImplementedError("write your pallas kernel here")


# speedup vs baseline: 3.4821x; 3.4821x over previous
import functools

import jax
import jax.numpy as jnp
from jax import lax
from jax.experimental import pallas as pl
from jax.experimental.pallas import tpu as pltpu
from jax.experimental.pallas import tpu_sc as plsc

N = 10000
E = 320000
D = 128
NC = 2
NS = 16
NW = NC * NS
EPW = E // NW
B = 80
NB = EPW // B
CH = 624
CHL = N - (NS - 1) * CH
NP = 10240

_MESH = plsc.VectorSubcoreMesh(
    core_axis_name="c", subcore_axis_name="s", num_cores=NC, num_subcores=NS)


@functools.partial(
    pl.kernel,
    out_type=jax.ShapeDtypeStruct((NC * 2 * NP,), jnp.float32),
    mesh=_MESH,
    scratch_types=[
        pltpu.VMEM_SHARED((NP,), jnp.float32),
        pltpu.VMEM_SHARED((NP,), jnp.float32),
        pltpu.VMEM((B,), jnp.int32),
        pltpu.VMEM((B,), jnp.int32),
        pltpu.VMEM((B,), jnp.float32),
    ],
)
def _deg_kernel(src_hbm, dst_hbm, zeros_hbm, deg_hbm,
                dego_s, degi_s, sidx_v, didx_v, ones_v):
    c = lax.axis_index("c")
    s = lax.axis_index("s")
    wid = s * NC + c
    for j in range(B // 16):
        ones_v[pl.ds(16 * j, 16)] = jnp.full((16,), 1.0, jnp.float32)

    @pl.when(s == 0)
    def _init():
        pltpu.sync_copy(zeros_hbm, dego_s)
        pltpu.sync_copy(zeros_hbm, degi_s)

    plsc.subcore_barrier()

    def body(i, carry):
        base = wid * EPW + i * B
        pltpu.sync_copy(src_hbm.at[pl.ds(base, B)], sidx_v)
        pltpu.sync_copy(dst_hbm.at[pl.ds(base, B)], didx_v)
        pltpu.sync_copy(ones_v, dego_s.at[sidx_v], add=True)
        pltpu.sync_copy(ones_v, degi_s.at[didx_v], add=True)
        return carry

    lax.fori_loop(0, NB, body, 0)
    plsc.subcore_barrier()

    @pl.when(s == 0)
    def _writeback():
        pltpu.sync_copy(dego_s, deg_hbm.at[pl.ds((c * 2 + 0) * NP, NP)])
        pltpu.sync_copy(degi_s, deg_hbm.at[pl.ds((c * 2 + 1) * NP, NP)])


@functools.partial(
    pl.kernel,
    out_type=jax.ShapeDtypeStruct((NC, N, D), jnp.float32),
    mesh=_MESH,
    scratch_types=[
        pltpu.VMEM_SHARED((N, D), jnp.float32),
        pltpu.VMEM((B,), jnp.int32),
        pltpu.VMEM((B,), jnp.int32),
        pltpu.VMEM((B, D), jnp.float32),
        pltpu.SemaphoreType.DMA,
    ],
)
def _agg_kernel(featsrc_hbm, src_hbm, dst_hbm, zrows_hbm, acc_hbm,
                acc_s, sidx_v, didx_v, rows_v, sem):
    c = lax.axis_index("c")
    s = lax.axis_index("s")
    wid = s * NC + c
    r0 = s * CH

    @pl.when(s < NS - 1)
    def _init_main():
        pltpu.sync_copy(zrows_hbm.at[pl.ds(0, CH)], acc_s.at[pl.ds(r0, CH)])

    @pl.when(s == NS - 1)
    def _init_last():
        pltpu.sync_copy(zrows_hbm, acc_s.at[pl.ds((NS - 1) * CH, CHL)])

    plsc.subcore_barrier()

    def body(i, carry):
        base = wid * EPW + i * B
        pltpu.sync_copy(src_hbm.at[pl.ds(base, B)], sidx_v)
        pltpu.sync_copy(dst_hbm.at[pl.ds(base, B)], didx_v)
        pltpu.async_copy(featsrc_hbm.at[sidx_v], rows_v, sem).wait()
        pltpu.sync_copy(rows_v, acc_s.at[didx_v], add=True)
        return carry

    lax.fori_loop(0, NB, body, 0)
    plsc.subcore_barrier()

    @pl.when(s < NS - 1)
    def _wb_main():
        pltpu.sync_copy(acc_s.at[pl.ds(r0, CH)], acc_hbm.at[c, pl.ds(r0, CH)])

    @pl.when(s == NS - 1)
    def _wb_last():
        pltpu.sync_copy(acc_s.at[pl.ds((NS - 1) * CH, CHL)],
                        acc_hbm.at[c, pl.ds((NS - 1) * CH, CHL)])


RB = 1000


def _scale_body(feat_ref, deg_ref, out_ref):
    d = deg_ref[0, 0] + deg_ref[1, 0]
    norm = lax.rsqrt(jnp.maximum(d, 1.0))
    out_ref[...] = feat_ref[...] * norm


_scale = pl.pallas_call(
    _scale_body,
    grid=(N // RB,),
    in_specs=[
        pl.BlockSpec((RB, D), lambda i: (i, 0)),
        pl.BlockSpec((NC, 2, RB, 1), lambda i: (0, 0, i, 0)),
    ],
    out_specs=pl.BlockSpec((RB, D), lambda i: (i, 0)),
    out_shape=jax.ShapeDtypeStruct((N, D), jnp.float32),
)


def _proj_body(acc_ref, w_ref, b_ref, deg_ref, out_ref):
    a = acc_ref[0] + acc_ref[1]
    y = lax.dot_general(a, w_ref[...], (((1,), (1,)), ((), ())),
                        preferred_element_type=jnp.float32)
    d = deg_ref[0, 1] + deg_ref[1, 1]
    norm = lax.rsqrt(jnp.maximum(d, 1.0))
    out_ref[...] = (y + b_ref[...]) * norm


_proj = pl.pallas_call(
    _proj_body,
    grid=(N // RB,),
    in_specs=[
        pl.BlockSpec((NC, RB, D), lambda i: (0, i, 0)),
        pl.BlockSpec((D, D), lambda i: (0, 0)),
        pl.BlockSpec((1, D), lambda i: (0, 0)),
        pl.BlockSpec((NC, 2, RB, 1), lambda i: (0, 0, i, 0)),
    ],
    out_specs=pl.BlockSpec((RB, D), lambda i: (i, 0)),
    out_shape=jax.ShapeDtypeStruct((N, D), jnp.float32),
)


def kernel(feat, edge_index, W, b):
    edge_index = edge_index.astype(jnp.int32)
    src = edge_index[0]
    dst = edge_index[1]
    zeros_col = jnp.zeros((NP,), jnp.float32)
    zeros_rows = jnp.zeros((CHL, D), jnp.float32)
    degs = _deg_kernel(src, dst, zeros_col).reshape(NC, 2, NP)[:, :, :N]
    degs = degs.reshape(NC, 2, N, 1)
    feat_src = _scale(feat, degs)
    acc = _agg_kernel(feat_src, src, dst, zeros_rows)
    return _proj(acc, W, b.reshape(1, D), degs)


# speedup vs baseline: 4.2730x; 1.2271x over previous
import functools

import jax
import jax.numpy as jnp
from jax import lax
from jax.experimental import pallas as pl
from jax.experimental.pallas import tpu as pltpu
from jax.experimental.pallas import tpu_sc as plsc

N = 10000
E = 320000
D = 128
NC = 2
NS = 16
NW = NC * NS
B = 128
PB = 2528
PE = PB * B
NBW = PB // NW
NP = 10240
NJ = N + 8
CH = 624
CHL = NJ - (NS - 1) * CH

_MESH = plsc.VectorSubcoreMesh(
    core_axis_name="c", subcore_axis_name="s", num_cores=NC, num_subcores=NS)


@functools.partial(
    pl.kernel,
    out_type=jax.ShapeDtypeStruct((NC * 2 * NP,), jnp.float32),
    mesh=_MESH,
    scratch_types=[
        pltpu.VMEM_SHARED((NP,), jnp.float32),
        pltpu.VMEM_SHARED((NP,), jnp.float32),
        pltpu.VMEM((NBW * B,), jnp.int32),
        pltpu.VMEM((NBW * B,), jnp.int32),
        pltpu.VMEM((B,), jnp.float32),
        pltpu.SemaphoreType.DMA,
        pltpu.SemaphoreType.DMA,
    ],
)
def _deg_kernel(src_hbm, dst_hbm, zeros_hbm, deg_hbm,
                dego_s, degi_s, sidx_v, didx_v, ones_v, sem0, sem1):
    c = lax.axis_index("c")
    s = lax.axis_index("s")
    wid = s * NC + c
    e0 = wid * NBW * B
    for j in range(B // 16):
        ones_v[pl.ds(16 * j, 16)] = jnp.full((16,), 1.0, jnp.float32)
    pltpu.async_copy(src_hbm.at[pl.ds(e0, NBW * B)], sidx_v, sem0).wait()
    pltpu.async_copy(dst_hbm.at[pl.ds(e0, NBW * B)], didx_v, sem1).wait()

    @pl.when(s == 0)
    def _init():
        pltpu.sync_copy(zeros_hbm, dego_s)
        pltpu.sync_copy(zeros_hbm, degi_s)

    plsc.subcore_barrier()

    def body(j, carry):
        o_copy = pltpu.make_async_copy(
            ones_v, dego_s.at[sidx_v.at[pl.ds(j * B, B)]], sem0)
        i_copy = pltpu.make_async_copy(
            ones_v, degi_s.at[didx_v.at[pl.ds(j * B, B)]], sem1)
        o_copy.start(add=True)
        i_copy.start(add=True)
        o_copy.wait()
        i_copy.wait()
        return carry

    lax.fori_loop(0, NBW, body, 0)
    plsc.subcore_barrier()

    @pl.when(s == 0)
    def _writeback():
        pltpu.sync_copy(dego_s, deg_hbm.at[pl.ds((c * 2 + 0) * NP, NP)])
        pltpu.sync_copy(degi_s, deg_hbm.at[pl.ds((c * 2 + 1) * NP, NP)])


@functools.partial(
    pl.kernel,
    out_type=jax.ShapeDtypeStruct((NC, N, D), jnp.float32),
    mesh=_MESH,
    scratch_types=[
        pltpu.VMEM_SHARED((NJ, D), jnp.float32),
        pltpu.VMEM((NBW * B,), jnp.int32),
        pltpu.VMEM((NBW * B,), jnp.int32),
        pltpu.VMEM((B, D), jnp.float32),
        pltpu.SemaphoreType.DMA,
        pltpu.SemaphoreType.DMA,
    ],
)
def _agg_kernel(featsrc_hbm, src_hbm, dst_hbm, zrows_hbm, acc_hbm,
                acc_s, sidx_v, didx_v, rows0_v, gsem0, gsem1):
    c = lax.axis_index("c")
    s = lax.axis_index("s")
    wid = s * NC + c
    r0 = s * CH
    e0 = wid * NBW * B
    pltpu.async_copy(src_hbm.at[pl.ds(e0, NBW * B)], sidx_v, gsem0)
    pltpu.async_copy(dst_hbm.at[pl.ds(e0, NBW * B)], didx_v, gsem1)

    @pl.when(s < NS - 1)
    def _init_main():
        pltpu.sync_copy(zrows_hbm.at[pl.ds(0, CH)], acc_s.at[pl.ds(r0, CH)])

    @pl.when(s == NS - 1)
    def _init_last():
        pltpu.sync_copy(zrows_hbm, acc_s.at[pl.ds((NS - 1) * CH, CHL)])

    pltpu.make_async_copy(src_hbm.at[pl.ds(e0, NBW * B)],
                          sidx_v, gsem0).wait()
    pltpu.make_async_copy(dst_hbm.at[pl.ds(e0, NBW * B)],
                          didx_v, gsem1).wait()
    plsc.subcore_barrier()

    def _gather(j, rows_v, gsem):
        return pltpu.make_async_copy(
            featsrc_hbm.at[sidx_v.at[pl.ds(j * B, B)]], rows_v, gsem)

    def body(j, carry):
        g = _gather(j, rows0_v, gsem0)
        g.start()
        g.wait()
        pltpu.sync_copy(rows0_v, acc_s.at[didx_v.at[pl.ds(j * B, B)]],
                        add=True)
        return carry

    lax.fori_loop(0, NBW, body, 0)

    plsc.subcore_barrier()

    @pl.when(s < NS - 1)
    def _wb_main():
        pltpu.sync_copy(acc_s.at[pl.ds(r0, CH)], acc_hbm.at[c, pl.ds(r0, CH)])

    @pl.when(s == NS - 1)
    def _wb_last():
        pltpu.sync_copy(acc_s.at[pl.ds((NS - 1) * CH, N - (NS - 1) * CH)],
                        acc_hbm.at[c, pl.ds((NS - 1) * CH, N - (NS - 1) * CH)])


RB = 1000


def _scale_body(feat_ref, deg_ref, out_ref):
    d = deg_ref[0, 0] + deg_ref[1, 0]
    norm = lax.rsqrt(jnp.maximum(d, 1.0))
    out_ref[...] = feat_ref[...] * norm


_scale = pl.pallas_call(
    _scale_body,
    grid=(N // RB,),
    in_specs=[
        pl.BlockSpec((RB, D), lambda i: (i, 0)),
        pl.BlockSpec((NC, 2, RB, 1), lambda i: (0, 0, i, 0)),
    ],
    out_specs=pl.BlockSpec((RB, D), lambda i: (i, 0)),
    out_shape=jax.ShapeDtypeStruct((N, D), jnp.float32),
)


def _proj_body(acc_ref, w_ref, b_ref, deg_ref, out_ref):
    a = acc_ref[0] + acc_ref[1]
    y = lax.dot_general(a, w_ref[...], (((1,), (1,)), ((), ())),
                        preferred_element_type=jnp.float32)
    d = deg_ref[0, 1] + deg_ref[1, 1]
    norm = lax.rsqrt(jnp.maximum(d, 1.0))
    out_ref[...] = (y + b_ref[...]) * norm


_proj = pl.pallas_call(
    _proj_body,
    grid=(N // RB,),
    in_specs=[
        pl.BlockSpec((NC, RB, D), lambda i: (0, i, 0)),
        pl.BlockSpec((D, D), lambda i: (0, 0)),
        pl.BlockSpec((1, D), lambda i: (0, 0)),
        pl.BlockSpec((NC, 2, RB, 1), lambda i: (0, 0, i, 0)),
    ],
    out_specs=pl.BlockSpec((RB, D), lambda i: (i, 0)),
    out_shape=jax.ShapeDtypeStruct((N, D), jnp.float32),
)


def kernel(feat, edge_index, W, b):
    edge_index = edge_index.astype(jnp.int32)
    src = edge_index[0]
    dst = edge_index[1]
    pad = jnp.full((PE - E,), N, jnp.int32)
    srcp = jnp.concatenate([src, jnp.zeros((PE - E,), jnp.int32)])
    srcd = jnp.concatenate([src, pad])
    dstp = jnp.concatenate([dst, pad])
    zeros_col = jnp.zeros((NP,), jnp.float32)
    zeros_rows = jnp.zeros((CHL, D), jnp.float32)
    degs = _deg_kernel(srcd, dstp, zeros_col).reshape(NC, 2, NP)[:, :, :N]
    degs = degs.reshape(NC, 2, N, 1)
    feat_src = _scale(feat, degs)
    acc = _agg_kernel(feat_src, srcp, dstp, zeros_rows)
    return _proj(acc, W, b.reshape(1, D), degs)


# speedup vs baseline: 4.7643x; 1.1150x over previous
import functools

import jax
import jax.numpy as jnp
from jax import lax
from jax.experimental import pallas as pl
from jax.experimental.pallas import tpu as pltpu
from jax.experimental.pallas import tpu_sc as plsc

N = 10000
E = 320000
D = 128
NC = 2
NS = 16
NW = NC * NS
B = 128
PB = 2528
PE = PB * B
NBW = PB // NW
RND = 40
NP = 10240
NJ = N + 8
CH = 624
CHL = NJ - (NS - 1) * CH

_MESH = plsc.VectorSubcoreMesh(
    core_axis_name="c", subcore_axis_name="s", num_cores=NC, num_subcores=NS)


@functools.partial(
    pl.kernel,
    out_type=jax.ShapeDtypeStruct((NC * 2 * NP,), jnp.float32),
    mesh=_MESH,
    scratch_types=[
        pltpu.VMEM_SHARED((NP,), jnp.float32),
        pltpu.VMEM_SHARED((NP,), jnp.float32),
        pltpu.VMEM((NBW * B,), jnp.int32),
        pltpu.VMEM((NBW * B,), jnp.int32),
        pltpu.VMEM((B,), jnp.float32),
        pltpu.SemaphoreType.DMA,
        pltpu.SemaphoreType.DMA,
    ],
)
def _deg_kernel(src_hbm, dst_hbm, zeros_hbm, deg_hbm,
                dego_s, degi_s, sidx_v, didx_v, ones_v, sem0, sem1):
    c = lax.axis_index("c")
    s = lax.axis_index("s")
    wid = s * NC + c
    e0 = wid * NBW * B
    for j in range(B // 16):
        ones_v[pl.ds(16 * j, 16)] = jnp.full((16,), 1.0, jnp.float32)
    pltpu.async_copy(src_hbm.at[pl.ds(e0, NBW * B)], sidx_v, sem0).wait()
    pltpu.async_copy(dst_hbm.at[pl.ds(e0, NBW * B)], didx_v, sem1).wait()

    @pl.when(s == 0)
    def _init():
        pltpu.sync_copy(zeros_hbm, dego_s)
        pltpu.sync_copy(zeros_hbm, degi_s)

    plsc.subcore_barrier()

    def body(j, carry):
        o_copy = pltpu.make_async_copy(
            ones_v, dego_s.at[sidx_v.at[pl.ds(j * B, B)]], sem0)
        i_copy = pltpu.make_async_copy(
            ones_v, degi_s.at[didx_v.at[pl.ds(j * B, B)]], sem1)
        o_copy.start(add=True)
        i_copy.start(add=True)
        o_copy.wait()
        i_copy.wait()
        return carry

    lax.fori_loop(0, NBW, body, 0)
    plsc.subcore_barrier()

    @pl.when(s == 0)
    def _writeback():
        pltpu.sync_copy(dego_s, deg_hbm.at[pl.ds((c * 2 + 0) * NP, NP)])
        pltpu.sync_copy(degi_s, deg_hbm.at[pl.ds((c * 2 + 1) * NP, NP)])


@functools.partial(
    pl.kernel,
    out_type=jax.ShapeDtypeStruct((NC, N, D), jnp.float32),
    mesh=_MESH,
    scratch_types=[
        pltpu.VMEM_SHARED((NJ, D), jnp.float32),
        pltpu.VMEM((NBW * B,), jnp.int32),
        pltpu.VMEM((RND * B,), jnp.int32),
        pltpu.VMEM((B, D), jnp.float32),
        pltpu.VMEM((B, D), jnp.float32),
        pltpu.SemaphoreType.DMA,
        pltpu.SemaphoreType.DMA,
    ],
)
def _agg_kernel(featsrc_hbm, src_hbm, dst_hbm, zrows_hbm, acc_hbm,
                acc_s, sidx_v, didx_v, rows0_v, rows1_v, gsem0, gsem1):
    c = lax.axis_index("c")
    s = lax.axis_index("s")
    wid = s * NC + c
    r0 = s * CH
    e0 = wid * NBW * B
    pltpu.async_copy(src_hbm.at[pl.ds(e0, NBW * B)], sidx_v, gsem0)
    pltpu.async_copy(dst_hbm.at[pl.ds(e0, RND * B)], didx_v, gsem1)

    @pl.when(s < NS - 1)
    def _init_main():
        pltpu.sync_copy(zrows_hbm.at[pl.ds(0, CH)], acc_s.at[pl.ds(r0, CH)])

    @pl.when(s == NS - 1)
    def _init_last():
        pltpu.sync_copy(zrows_hbm, acc_s.at[pl.ds((NS - 1) * CH, CHL)])

    pltpu.make_async_copy(src_hbm.at[pl.ds(e0, NBW * B)],
                          sidx_v, gsem0).wait()
    pltpu.make_async_copy(dst_hbm.at[pl.ds(e0, RND * B)],
                          didx_v, gsem1).wait()
    plsc.subcore_barrier()

    def _gather(j, rows_v, gsem):
        return pltpu.make_async_copy(
            featsrc_hbm.at[sidx_v.at[pl.ds(j * B, B)]], rows_v, gsem)

    def _scatter(rows_v, jr):
        pltpu.sync_copy(rows_v, acc_s.at[didx_v.at[pl.ds(jr * B, B)]],
                        add=True)

    def _pair(base):
        def pair(k, carry):
            j0 = base + 2 * k
            _gather(j0, rows0_v, gsem0).wait()
            _gather(j0 + 1, rows1_v, gsem1).start()
            _scatter(rows0_v, j0 - base)
            _gather(j0 + 1, rows1_v, gsem1).wait()
            _gather(j0 + 2, rows0_v, gsem0).start()
            _scatter(rows1_v, j0 + 1 - base)
            return carry
        return pair

    _gather(0, rows0_v, gsem0).start()
    lax.fori_loop(0, RND // 2, _pair(0), 0)
    pltpu.sync_copy(dst_hbm.at[pl.ds(e0 + RND * B, (NBW - RND) * B)],
                    didx_v.at[pl.ds(0, (NBW - RND) * B)])
    lax.fori_loop(0, (NBW - RND - 1) // 2, _pair(RND), 0)
    _gather(NBW - 1, rows0_v, gsem0).wait()
    _scatter(rows0_v, NBW - 1 - RND)

    plsc.subcore_barrier()

    @pl.when(s < NS - 1)
    def _wb_main():
        pltpu.sync_copy(acc_s.at[pl.ds(r0, CH)], acc_hbm.at[c, pl.ds(r0, CH)])

    @pl.when(s == NS - 1)
    def _wb_last():
        pltpu.sync_copy(acc_s.at[pl.ds((NS - 1) * CH, N - (NS - 1) * CH)],
                        acc_hbm.at[c, pl.ds((NS - 1) * CH, N - (NS - 1) * CH)])


RB = 1000


def _scale_body(feat_ref, deg_ref, out_ref):
    d = deg_ref[0, 0] + deg_ref[1, 0]
    norm = lax.rsqrt(jnp.maximum(d, 1.0))
    out_ref[...] = feat_ref[...] * norm


_scale = pl.pallas_call(
    _scale_body,
    grid=(N // RB,),
    in_specs=[
        pl.BlockSpec((RB, D), lambda i: (i, 0)),
        pl.BlockSpec((NC, 2, RB, 1), lambda i: (0, 0, i, 0)),
    ],
    out_specs=pl.BlockSpec((RB, D), lambda i: (i, 0)),
    out_shape=jax.ShapeDtypeStruct((N, D), jnp.float32),
)


def _proj_body(acc_ref, w_ref, b_ref, deg_ref, out_ref):
    a = acc_ref[0] + acc_ref[1]
    y = lax.dot_general(a, w_ref[...], (((1,), (1,)), ((), ())),
                        preferred_element_type=jnp.float32)
    d = deg_ref[0, 1] + deg_ref[1, 1]
    norm = lax.rsqrt(jnp.maximum(d, 1.0))
    out_ref[...] = (y + b_ref[...]) * norm


_proj = pl.pallas_call(
    _proj_body,
    grid=(N // RB,),
    in_specs=[
        pl.BlockSpec((NC, RB, D), lambda i: (0, i, 0)),
        pl.BlockSpec((D, D), lambda i: (0, 0)),
        pl.BlockSpec((1, D), lambda i: (0, 0)),
        pl.BlockSpec((NC, 2, RB, 1), lambda i: (0, 0, i, 0)),
    ],
    out_specs=pl.BlockSpec((RB, D), lambda i: (i, 0)),
    out_shape=jax.ShapeDtypeStruct((N, D), jnp.float32),
)


def kernel(feat, edge_index, W, b):
    edge_index = edge_index.astype(jnp.int32)
    src = edge_index[0]
    dst = edge_index[1]
    pad = jnp.full((PE - E,), N, jnp.int32)
    srcp = jnp.concatenate([src, jnp.zeros((PE - E,), jnp.int32)])
    srcd = jnp.concatenate([src, pad])
    dstp = jnp.concatenate([dst, pad])
    zeros_col = jnp.zeros((NP,), jnp.float32)
    zeros_rows = jnp.zeros((CHL, D), jnp.float32)
    degs = _deg_kernel(srcd, dstp, zeros_col).reshape(NC, 2, NP)[:, :, :N]
    degs = degs.reshape(NC, 2, N, 1)
    feat_src = _scale(feat, degs)
    acc = _agg_kernel(feat_src, srcp, dstp, zeros_rows)
    return _proj(acc, W, b.reshape(1, D), degs)
